# baseline (device time: 114322 ns/iter reference)
import functools

import jax
import jax.numpy as jnp
from jax import lax
from jax.experimental import pallas as pl
from jax.experimental.pallas import tpu as pltpu

N_DEV = 16
SQ = 1024
DM = 1024
H = 8
DH = 128
BLK = 64
QR = 256
HC = 512
ZR = 64
SCALE = 0.08838834764831843


def _m4(v):
    return lax.rem(v + 16, 4)


def kernel(x, Wq, K_ext, V_ext, Wo):
    my = lax.axis_index("i")
    x2 = x[0]
    k_my = lax.dynamic_slice(K_ext, (0, 0, my * H, 0), (1, SQ, H, DH))[0]
    v_my = lax.dynamic_slice(V_ext, (0, 0, my * H, 0), (1, SQ, H, DH))[0]

    def body(x_ref, wq_ref, k_ref, v_ref, wo_ref, out_ref,
             ctx_ref, bufA, bufB,
             semA_send, semA_recv,
             semBr_send, semBr_recv, semBa_send, semBa_recv,
             semC_send, semC_recv):
        my_pos = lax.axis_index("i")
        p = lax.rem(my_pos, 4)
        zz = my_pos // 4
        base = zz * 4
        rightP = base + _m4(p + 1)
        leftP = base + _m4(p - 1)
        zpeers = [lax.rem(zz + dz, 4) * 4 + p for dz in (1, 2, 3)]

        peers = [leftP, rightP] + zpeers
        barrier_sem = pltpu.get_barrier_semaphore()
        for nbr in peers:
            pl.semaphore_signal(barrier_sem, inc=1, device_id=(nbr,),
                                device_id_type=pl.DeviceIdType.MESH)
        pl.semaphore_wait(barrier_sem, len(peers))

        q = jnp.dot(x_ref[...], wq_ref[...],
                    preferred_element_type=jnp.float32)
        row = lax.broadcasted_iota(jnp.int32, (SQ, SQ), 0) // BLK
        col = lax.broadcasted_iota(jnp.int32, (SQ, SQ), 1) // BLK
        mask = (row == col) | (col == 0) | (lax.rem(row + col, 3) == 0)
        for h in range(H):
            qh = q[:, h * DH:(h + 1) * DH]
            kh = k_ref[:, h, :]
            vh = v_ref[:, h, :]
            s = lax.dot_general(qh, kh, (((1,), (1,)), ((), ())),
                                preferred_element_type=jnp.float32)
            s = jnp.where(mask, s * SCALE, -1e9)
            m = jnp.max(s, axis=1, keepdims=True)
            w = jnp.exp(s - m)
            w = w / jnp.sum(w, axis=1, keepdims=True)
            ctx_ref[:, h * DH:(h + 1) * DH] = jnp.dot(
                w, vh, preferred_element_type=jnp.float32)
        out_ref[...] = jnp.dot(ctx_ref[...], wo_ref[...],
                               preferred_element_type=jnp.float32)

        for s_i in range(3):
            sq0 = _m4(p - s_i)
            sq1 = _m4(p + s_i)
            d0 = pltpu.make_async_remote_copy(
                src_ref=out_ref.at[pl.ds(sq0 * QR, QR), pl.ds(0, HC)],
                dst_ref=bufA.at[0, s_i],
                send_sem=semA_send.at[0, s_i], recv_sem=semA_recv.at[0, s_i],
                device_id=(rightP,), device_id_type=pl.DeviceIdType.MESH)
            d1 = pltpu.make_async_remote_copy(
                src_ref=out_ref.at[pl.ds(sq1 * QR, QR), pl.ds(HC, HC)],
                dst_ref=bufA.at[1, s_i],
                send_sem=semA_send.at[1, s_i], recv_sem=semA_recv.at[1, s_i],
                device_id=(leftP,), device_id_type=pl.DeviceIdType.MESH)
            d0.start()
            d1.start()
            d0.wait()
            d1.wait()
            rq0 = _m4(p - 1 - s_i)
            rq1 = _m4(p + 1 + s_i)
            r0 = pl.ds(rq0 * QR, QR)
            r1 = pl.ds(rq1 * QR, QR)
            out_ref[r0, 0:HC] = out_ref[r0, 0:HC] + bufA[0, s_i]
            out_ref[r1, HC:2 * HC] = out_ref[r1, HC:2 * HC] + bufA[1, s_i]

        own0 = _m4(p + 1)
        own1 = _m4(p - 1)

        rs_descs = []
        for dz in (1, 2, 3):
            zt = lax.rem(zz + dz, 4)
            tgt = zt * 4 + p
            slot = 3 - dz
            for piece, (oq, c0) in enumerate(((own0, 0), (own1, HC))):
                d = pltpu.make_async_remote_copy(
                    src_ref=out_ref.at[pl.ds(oq * QR + zt * ZR, ZR),
                                       pl.ds(c0, HC)],
                    dst_ref=bufB.at[slot, piece],
                    send_sem=semBr_send.at[slot, piece],
                    recv_sem=semBr_recv.at[slot, piece],
                    device_id=(tgt,), device_id_type=pl.DeviceIdType.MESH)
                d.start()
                rs_descs.append(d)
        for slot in range(3):
            for piece in range(2):
                rd = pltpu.make_async_remote_copy(
                    src_ref=bufB.at[slot, piece], dst_ref=bufB.at[slot, piece],
                    send_sem=semBr_send.at[slot, piece],
                    recv_sem=semBr_recv.at[slot, piece],
                    device_id=(my_pos,), device_id_type=pl.DeviceIdType.MESH)
                rd.wait_recv()
        for d in rs_descs:
            d.wait_send()
        s0 = pl.ds(own0 * QR + zz * ZR, ZR)
        s1 = pl.ds(own1 * QR + zz * ZR, ZR)
        out_ref[s0, 0:HC] = (out_ref[s0, 0:HC]
                             + bufB[0, 0] + bufB[1, 0] + bufB[2, 0])
        out_ref[s1, HC:2 * HC] = (out_ref[s1, HC:2 * HC]
                                  + bufB[0, 1] + bufB[1, 1] + bufB[2, 1])

        ag_descs = []
        for dz in (1, 2, 3):
            zt = lax.rem(zz + dz, 4)
            tgt = zt * 4 + p
            slot = 3 - dz
            for piece, (oq, c0) in enumerate(((own0, 0), (own1, HC))):
                strip = out_ref.at[pl.ds(oq * QR + zz * ZR, ZR), pl.ds(c0, HC)]
                d = pltpu.make_async_remote_copy(
                    src_ref=strip, dst_ref=strip,
                    send_sem=semBa_send.at[slot, piece],
                    recv_sem=semBa_recv.at[slot, piece],
                    device_id=(tgt,), device_id_type=pl.DeviceIdType.MESH)
                d.start()
                ag_descs.append(d)
        for slot in range(3):
            for piece in range(2):
                c0 = 0 if piece == 0 else HC
                oq = own0 if piece == 0 else own1
                rd = pltpu.make_async_remote_copy(
                    src_ref=out_ref.at[pl.ds(oq * QR + zz * ZR, ZR),
                                       pl.ds(c0, HC)],
                    dst_ref=out_ref.at[pl.ds(oq * QR + zz * ZR, ZR),
                                       pl.ds(c0, HC)],
                    send_sem=semBa_send.at[slot, piece],
                    recv_sem=semBa_recv.at[slot, piece],
                    device_id=(my_pos,), device_id_type=pl.DeviceIdType.MESH)
                rd.wait_recv()
        for d in ag_descs:
            d.wait_send()

        for t in range(3):
            sq0 = _m4(p + 1 - t)
            sq1 = _m4(p - 1 + t)
            src0 = out_ref.at[pl.ds(sq0 * QR, QR), pl.ds(0, HC)]
            src1 = out_ref.at[pl.ds(sq1 * QR, QR), pl.ds(HC, HC)]
            d0 = pltpu.make_async_remote_copy(
                src_ref=src0, dst_ref=src0,
                send_sem=semC_send.at[0, t], recv_sem=semC_recv.at[0, t],
                device_id=(rightP,), device_id_type=pl.DeviceIdType.MESH)
            d1 = pltpu.make_async_remote_copy(
                src_ref=src1, dst_ref=src1,
                send_sem=semC_send.at[1, t], recv_sem=semC_recv.at[1, t],
                device_id=(leftP,), device_id_type=pl.DeviceIdType.MESH)
            d0.start()
            d1.start()
            d0.wait()
            d1.wait()

        @functools.partial(pl.run_scoped, sem2=pltpu.SemaphoreType.REGULAR)
        def _(sem2):
            for nbr in peers:
                pl.semaphore_signal(sem2, inc=1, device_id=(nbr,),
                                    device_id_type=pl.DeviceIdType.MESH)
            pl.semaphore_wait(sem2, len(peers))

    out = pl.pallas_call(
        body,
        out_shape=jax.ShapeDtypeStruct((SQ, DM), jnp.float32),
        in_specs=[pl.BlockSpec(memory_space=pltpu.VMEM)] * 5,
        out_specs=pl.BlockSpec(memory_space=pltpu.VMEM),
        scratch_shapes=[
            pltpu.VMEM((SQ, H * DH), jnp.float32),
            pltpu.VMEM((2, 3, QR, HC), jnp.float32),
            pltpu.VMEM((3, 2, ZR, HC), jnp.float32),
            pltpu.SemaphoreType.DMA((2, 3)),
            pltpu.SemaphoreType.DMA((2, 3)),
            pltpu.SemaphoreType.DMA((3, 2)),
            pltpu.SemaphoreType.DMA((3, 2)),
            pltpu.SemaphoreType.DMA((3, 2)),
            pltpu.SemaphoreType.DMA((3, 2)),
            pltpu.SemaphoreType.DMA((2, 3)),
            pltpu.SemaphoreType.DMA((2, 3)),
        ],
        compiler_params=pltpu.CompilerParams(collective_id=0),
    )(x2, Wq, k_my, v_my, Wo)
    return out[None]
